# baseline (speedup 1.0000x reference)
import functools
import jax
import jax.numpy as jnp
from jax import lax
from jax.experimental import pallas as pl
from jax.experimental.pallas import tpu as pltpu
from jax.experimental.pallas import tpu_sc as plsc

_NW = 32
_CHUNK = 2048


def _table_body(emb_ref, w_ref, b_ref, t_ref):
    t_ref[...] = lax.dot_general(
        emb_ref[...], w_ref[...], (((1,), (1,)), ((), ())),
        preferred_element_type=jnp.float32) + b_ref[...]


def _make_table(emb, W, b):
    return pl.pallas_call(
        _table_body,
        out_shape=jax.ShapeDtypeStruct((16, 16), jnp.float32),
    )(emb, W, b.reshape(1, 16))


def _sc_gather(table_flat, ids_flat):
    n = ids_flat.shape[0]
    per_w = n // _NW
    chunks = per_w // _CHUNK
    out_bytes = _CHUNK * 16
    mesh = plsc.VectorSubcoreMesh(core_axis_name="c", subcore_axis_name="s")

    @functools.partial(
        pl.kernel, mesh=mesh,
        out_type=jax.ShapeDtypeStruct((n * 16,), jnp.float32),
        scratch_types=[
            pltpu.VMEM((256,), jnp.float32),
            pltpu.VMEM((_CHUNK,), jnp.int32),
            pltpu.VMEM((_CHUNK * 16,), jnp.float32),
            pltpu.VMEM((_CHUNK * 16,), jnp.float32),
            pltpu.SemaphoreType.DMA,
            pltpu.SemaphoreType.DMA,
        ],
        compiler_params=pltpu.CompilerParams(needs_layout_passes=False),
    )
    def k(table_hbm, ids_hbm, out_hbm, table_v, ids_v, rows0, rows1,
          sem0, sem1):
        wid = lax.axis_index("s") * 2 + lax.axis_index("c")
        tok_base = wid * per_w
        pltpu.sync_copy(table_hbm, table_v)
        lane16 = lax.iota(jnp.int32, 16) * 16

        def do_chunk(c, rows_v, sem):
            tok0 = tok_base + c * _CHUNK
            pltpu.sync_copy(ids_hbm.at[pl.ds(tok0, _CHUNK)], ids_v)

            @pl.loop(0, _CHUNK // 16, unroll=2)
            def _(g):
                ids16 = ids_v[pl.ds(g * 16, 16)]
                a = ids16 * 16
                ob = lane16 + g * 256
                vs = [plsc.load_gather(table_v, [a + e]) for e in range(16)]
                for e in range(16):
                    plsc.store_scatter(rows_v, [ob + e], vs[e])

            pltpu.async_copy(
                rows_v, out_hbm.at[pl.ds(tok0 * 16, out_bytes)], sem)

        def wait_out(rows_v, sem):
            pltpu.make_async_copy(
                rows_v, out_hbm.at[pl.ds(0, out_bytes)], sem).wait()

        do_chunk(0, rows0, sem0)
        do_chunk(1, rows1, sem1)

        @pl.loop(2, chunks, step=2)
        def _(c):
            wait_out(rows0, sem0)
            do_chunk(c, rows0, sem0)
            wait_out(rows1, sem1)
            do_chunk(c + 1, rows1, sem1)

        wait_out(rows0, sem0)
        wait_out(rows1, sem1)

    return k(table_flat, ids_flat)


def kernel(input_ids, emb, W, b):
    B, T = input_ids.shape
    n = B * T
    table = _make_table(emb, W, b).reshape(256)
    out = _sc_gather(table, input_ids.reshape(n))
    return out.reshape(B, T, 16)


# speedup vs baseline: 11.5427x; 11.5427x over previous
import jax
import jax.numpy as jnp
from jax import lax
from jax.experimental import pallas as pl


def _body(ids_ref, emb_ref, w_ref, b_ref, out_ref):
    f32 = jnp.float32
    tableT = lax.dot_general(
        w_ref[...], emb_ref[...], (((1,), (1,)), ((), ())),
        preferred_element_type=f32) + b_ref[...]

    pm = lax.broadcasted_iota(jnp.int32, (128, 16), 0) % 16
    ei = lax.broadcasted_iota(jnp.int32, (128, 16), 1)
    left = (pm == ei).astype(f32)
    vj = lax.broadcasted_iota(jnp.int32, (16, 128), 0)
    qm = lax.broadcasted_iota(jnp.int32, (16, 128), 1) % 16
    right = (vj == qm).astype(f32)
    tiled = jnp.dot(jnp.dot(left, tableT, preferred_element_type=f32),
                    right, preferred_element_type=f32)
    pg = lax.broadcasted_iota(jnp.int32, (128, 128), 0) // 16
    qg = lax.broadcasted_iota(jnp.int32, (128, 128), 1) // 16
    bigT = jnp.where(pg == qg, tiled, 0.0)

    gi = lax.broadcasted_iota(jnp.int32, (128, 8), 0) // 16
    gj = lax.broadcasted_iota(jnp.int32, (128, 8), 1)
    rep = (gi == gj).astype(f32)
    idsrep = jnp.dot(rep, ids_ref[...].astype(f32),
                     preferred_element_type=f32)
    viota = (lax.broadcasted_iota(jnp.int32, idsrep.shape, 0) % 16).astype(f32)
    oh = (idsrep == viota).astype(f32)
    out_ref[...] = jnp.dot(bigT, oh, preferred_element_type=f32)


def kernel(input_ids, emb, W, b):
    B, T = input_ids.shape
    idsT = input_ids.T
    TB, NB = 8, 2048
    outT = pl.pallas_call(
        _body,
        grid=(T // TB, B // NB),
        in_specs=[
            pl.BlockSpec((TB, NB), lambda i, j: (i, j)),
            pl.BlockSpec((16, 16), lambda i, j: (0, 0)),
            pl.BlockSpec((16, 16), lambda i, j: (0, 0)),
            pl.BlockSpec((16, 1), lambda i, j: (0, 0)),
        ],
        out_specs=pl.BlockSpec((TB * 16, NB), lambda i, j: (i, j)),
        out_shape=jax.ShapeDtypeStruct((T * 16, B), jnp.float32),
    )(idsT, emb, W, b.reshape(16, 1))
    return jnp.transpose(outT.reshape(T, 16, B), (2, 0, 1))


# speedup vs baseline: 12.0150x; 1.0409x over previous
import functools
import jax
import jax.numpy as jnp
from jax import lax
from jax.experimental import pallas as pl
from jax.experimental.pallas import tpu as pltpu
from jax.experimental.pallas import tpu_sc as plsc

_NW = 32
_TG = 8
_EG = 4


def _table_body(emb_ref, w_ref, b_ref, t_ref):
    t_ref[...] = lax.dot_general(
        w_ref[...], emb_ref[...], (((1,), (1,)), ((), ())),
        preferred_element_type=jnp.float32) + b_ref[...]


def _make_table_t(emb, W, b):
    return pl.pallas_call(
        _table_body,
        out_shape=jax.ShapeDtypeStruct((16, 16), jnp.float32),
    )(emb, W, b.reshape(16, 1))


def _sc_gather(table_flat, idsT):
    T, B = idsT.shape
    t_per = T // _TG
    mesh = plsc.VectorSubcoreMesh(core_axis_name="c", subcore_axis_name="s")

    @functools.partial(
        pl.kernel, mesh=mesh,
        out_type=jax.ShapeDtypeStruct((T * 16, B), jnp.float32),
        scratch_types=[
            pltpu.VMEM((256,), jnp.float32),
            pltpu.VMEM((B,), jnp.int32),
            pltpu.VMEM((B,), jnp.float32),
            pltpu.VMEM((B,), jnp.float32),
            pltpu.SemaphoreType.DMA,
            pltpu.SemaphoreType.DMA,
        ],
        compiler_params=pltpu.CompilerParams(needs_layout_passes=False),
    )
    def k(table_hbm, ids_hbm, out_hbm, table_v, ids_v, rows0, rows1,
          sem0, sem1):
        wid = lax.axis_index("s") * 2 + lax.axis_index("c")
        t0 = (wid // _EG) * t_per
        e0 = (wid % _EG) * _EG
        pltpu.sync_copy(table_hbm, table_v)
        bufs = (rows0, rows1)
        sems = (sem0, sem1)

        def fill_row(e, rows_v):
            e16 = (e0 + e) * 16

            @pl.loop(0, B // 128, unroll=1)
            def _(gb):
                idss = [ids_v[pl.ds((gb * 8 + k) * 16, 16)]
                        for k in range(8)]
                vs = [plsc.load_gather(table_v, [idss[k] + e16])
                      for k in range(8)]
                for k in range(8):
                    rows_v[pl.ds((gb * 8 + k) * 16, 16)] = vs[k]

        def start_out(t, e, rows_v, sem):
            r = t * 16 + (e0 + e)
            pltpu.async_copy(rows_v, out_hbm.at[r], sem)

        def wait_out(rows_v, sem):
            pltpu.make_async_copy(rows_v, out_hbm.at[0], sem).wait()

        pltpu.sync_copy(ids_hbm.at[t0], ids_v)
        for e in range(_EG):
            if e >= 2:
                wait_out(bufs[e % 2], sems[e % 2])
            fill_row(e, bufs[e % 2])
            start_out(t0, e, bufs[e % 2], sems[e % 2])

        @pl.loop(t0 + 1, t0 + t_per)
        def _(t):
            pltpu.sync_copy(ids_hbm.at[t], ids_v)
            for e in range(_EG):
                wait_out(bufs[e % 2], sems[e % 2])
                fill_row(e, bufs[e % 2])
                start_out(t, e, bufs[e % 2], sems[e % 2])

        wait_out(rows0, sem0)
        wait_out(rows1, sem1)

    return k(table_flat, idsT)


def kernel(input_ids, emb, W, b):
    B, T = input_ids.shape
    tableT = _make_table_t(emb, W, b).reshape(256)
    outT = _sc_gather(tableT, input_ids.T)
    return jnp.transpose(outT.reshape(T, 16, B), (2, 0, 1))
